# baseline (speedup 1.0000x reference)
import functools

import jax
import jax.numpy as jnp
from jax import lax
from jax.experimental import pallas as pl
from jax.experimental.pallas import tpu as pltpu
from jax.experimental.pallas import tpu_sc as plsc

B = 16384
D = 64
NC, NS = 2, 16
NW = NC * NS
BPW = B // NW

_sc_mesh = plsc.VectorSubcoreMesh(core_axis_name="c", subcore_axis_name="s")


@functools.partial(
    pl.kernel,
    mesh=_sc_mesh,
    out_type=jax.ShapeDtypeStruct((B, 2 * D), jnp.bfloat16),
    scratch_types=[
        pltpu.VMEM((BPW,), jnp.int32),
        pltpu.VMEM((BPW,), jnp.int32),
        pltpu.VMEM((BPW, D), jnp.bfloat16),
        pltpu.VMEM((BPW, D), jnp.bfloat16),
        pltpu.SemaphoreType.DMA,
        pltpu.SemaphoreType.DMA,
    ],
    compiler_params=pltpu.CompilerParams(use_tc_tiling_on_sc=False),
)
def _sc_gather(uid_hbm, iid_hbm, utab_hbm, itab_hbm, out_hbm,
               uidx_v, iidx_v, urows_v, irows_v, usem, isem):
    wid = lax.axis_index("s") * NC + lax.axis_index("c")
    base = wid * BPW
    pltpu.sync_copy(uid_hbm.at[pl.ds(base, BPW)], uidx_v)
    ucp = pltpu.async_copy(utab_hbm.at[uidx_v], urows_v, usem)
    pltpu.sync_copy(iid_hbm.at[pl.ds(base, BPW)], iidx_v)
    icp = pltpu.async_copy(itab_hbm.at[iidx_v], irows_v, isem)
    ucp.wait()
    pltpu.sync_copy(urows_v, out_hbm.at[pl.ds(base, BPW), pl.ds(0, D)])
    icp.wait()
    pltpu.sync_copy(irows_v, out_hbm.at[pl.ds(base, BPW), pl.ds(D, D)])


BM = 8192
NB = B // BM


def _mlp_body(x_ref, w1t_ref, b1_ref, w2t_ref, b2_ref, w3_ref, b3_ref, o_ref):
    h = jnp.dot(x_ref[...], w1t_ref[...], preferred_element_type=jnp.float32)
    h = jnp.maximum(h + b1_ref[...], 0.0)
    h2 = jnp.dot(h.astype(jnp.bfloat16), w2t_ref[...],
                 preferred_element_type=jnp.float32)
    h2 = jnp.maximum(h2 + b2_ref[...], 0.0)
    o_ref[...] = jnp.sum(h2 * w3_ref[...], axis=1) + b3_ref[0, 0]


def _mlp(x, w1_t, b1, w2_t, b2, w3, b3):
    return pl.pallas_call(
        _mlp_body,
        grid=(NB,),
        in_specs=[
            pl.BlockSpec((BM, 2 * D), lambda j: (j, 0)),
            pl.BlockSpec((2 * D, 128), lambda j: (0, 0)),
            pl.BlockSpec((1, 128), lambda j: (0, 0)),
            pl.BlockSpec((128, D), lambda j: (0, 0)),
            pl.BlockSpec((1, D), lambda j: (0, 0)),
            pl.BlockSpec((1, D), lambda j: (0, 0)),
            pl.BlockSpec((1, 1), lambda j: (0, 0)),
        ],
        out_specs=pl.BlockSpec((BM,), lambda j: (j,)),
        out_shape=jax.ShapeDtypeStruct((B,), jnp.float32),
    )(x, w1_t, b1, w2_t, b2, w3, b3)


def kernel(user_ids, item_ids, user_table, item_table, W1, b1, W2, b2, W3, b3):
    x = _sc_gather(user_ids.astype(jnp.int32), item_ids.astype(jnp.int32),
                   user_table.astype(jnp.bfloat16),
                   item_table.astype(jnp.bfloat16))
    out = _mlp(x, W1.T.astype(jnp.bfloat16), b1.reshape(1, 128),
               W2.T.astype(jnp.bfloat16), b2.reshape(1, D),
               W3.reshape(1, D), b3.reshape(1, 1))
    return out


# speedup vs baseline: 1.4207x; 1.4207x over previous
import functools

import jax
import jax.numpy as jnp
from jax import lax
from jax.experimental import pallas as pl
from jax.experimental.pallas import tpu as pltpu
from jax.experimental.pallas import tpu_sc as plsc

B = 16384
D = 64
NC, NS = 2, 16
NW = NC * NS
BPW = B // NW

_sc_mesh = plsc.VectorSubcoreMesh(core_axis_name="c", subcore_axis_name="s")


@functools.partial(
    pl.kernel,
    mesh=_sc_mesh,
    out_type=jax.ShapeDtypeStruct((B, 2 * D), jnp.float32),
    scratch_types=[
        pltpu.VMEM((BPW,), jnp.int32),
        pltpu.VMEM((BPW,), jnp.int32),
        pltpu.VMEM((BPW, D), jnp.float32),
        pltpu.VMEM((BPW, D), jnp.float32),
        pltpu.SemaphoreType.DMA,
        pltpu.SemaphoreType.DMA,
    ],
    compiler_params=pltpu.CompilerParams(use_tc_tiling_on_sc=False),
)
def _sc_gather(uid_hbm, iid_hbm, utab_hbm, itab_hbm, out_hbm,
               uidx_v, iidx_v, urows_v, irows_v, usem, isem):
    wid = lax.axis_index("s") * NC + lax.axis_index("c")
    base = wid * BPW
    pltpu.sync_copy(uid_hbm.at[pl.ds(base, BPW)], uidx_v)
    ucp = pltpu.async_copy(utab_hbm.at[uidx_v], urows_v, usem)
    pltpu.sync_copy(iid_hbm.at[pl.ds(base, BPW)], iidx_v)
    icp = pltpu.async_copy(itab_hbm.at[iidx_v], irows_v, isem)
    ucp.wait()
    pltpu.sync_copy(urows_v, out_hbm.at[pl.ds(base, BPW), pl.ds(0, D)])
    icp.wait()
    pltpu.sync_copy(irows_v, out_hbm.at[pl.ds(base, BPW), pl.ds(D, D)])


BM = 8192
NB = B // BM


def _mlp_body(x_ref, w1t_ref, b1_ref, w2t_ref, b2_ref, w3_ref, b3_ref, o_ref):
    x16 = x_ref[...].astype(jnp.bfloat16)
    h = jnp.dot(x16, w1t_ref[...], preferred_element_type=jnp.float32)
    h = jnp.maximum(h + b1_ref[...], 0.0)
    h2 = jnp.dot(h.astype(jnp.bfloat16), w2t_ref[...],
                 preferred_element_type=jnp.float32)
    h2 = jnp.maximum(h2 + b2_ref[...], 0.0)
    o_ref[...] = jnp.sum(h2 * w3_ref[...], axis=1) + b3_ref[0, 0]


def _mlp(x, w1_t, b1, w2_t, b2, w3, b3):
    return pl.pallas_call(
        _mlp_body,
        grid=(NB,),
        in_specs=[
            pl.BlockSpec((BM, 2 * D), lambda j: (j, 0)),
            pl.BlockSpec((2 * D, 128), lambda j: (0, 0)),
            pl.BlockSpec((1, 128), lambda j: (0, 0)),
            pl.BlockSpec((128, D), lambda j: (0, 0)),
            pl.BlockSpec((1, D), lambda j: (0, 0)),
            pl.BlockSpec((1, D), lambda j: (0, 0)),
            pl.BlockSpec((1, 1), lambda j: (0, 0)),
        ],
        out_specs=pl.BlockSpec((BM,), lambda j: (j,)),
        out_shape=jax.ShapeDtypeStruct((B,), jnp.float32),
    )(x, w1_t, b1, w2_t, b2, w3, b3)


def kernel(user_ids, item_ids, user_table, item_table, W1, b1, W2, b2, W3, b3):
    x = _sc_gather(user_ids.astype(jnp.int32), item_ids.astype(jnp.int32),
                   user_table, item_table)
    out = _mlp(x, W1.T.astype(jnp.bfloat16), b1.reshape(1, 128),
               W2.T.astype(jnp.bfloat16), b2.reshape(1, D),
               W3.reshape(1, D), b3.reshape(1, 1))
    return out
